# baseline (speedup 1.0000x reference)
import functools
import math

import jax
import jax.numpy as jnp
from jax import lax
from jax.experimental import pallas as pl
from jax.experimental.pallas import tpu as pltpu
from jax.experimental.pallas import tpu_sc as plsc

_LOG2PI = math.log(2.0 * math.pi)


def _tc_body(x_ref, wa_ref, wb_ref, const_ref, c2_ref, out_ref):
    x = x_ref[...]
    x2 = x * x
    C = c2_ref.shape[1]
    CK = const_ref.shape[1]
    K = CK // C
    z1 = jnp.dot(x.astype(jnp.bfloat16), wa_ref[...],
                 preferred_element_type=jnp.float32)
    z2 = jnp.dot(x2.astype(jnp.bfloat16), wb_ref[...],
                 preferred_element_type=jnp.float32)
    scores = z2[:, CK:CK + 1] + z1[:, CK:] + c2_ref[...]
    lp = z1[:, :CK] + z2[:, :CK] + const_ref[...]
    lpt = lp.T
    ms, ss = [], []
    for c in range(C):
        seg = lpt[c * K:(c + 1) * K, :]
        m = jnp.max(seg, axis=0, keepdims=True)
        ms.append(m)
        ss.append(jnp.sum(jnp.exp(seg - m), axis=0, keepdims=True))
    lls_t = (jnp.log(jnp.concatenate(ss, axis=0))
             + jnp.concatenate(ms, axis=0))
    out_ref[...] = jnp.concatenate([scores.T, lls_t], axis=0)


def _tc_stage(x, w_a, w_b, const_row, c2_row, tb):
    B, D = x.shape
    CKC = w_a.shape[1]
    C = c2_row.shape[1]
    CK = CKC - C
    return pl.pallas_call(
        _tc_body,
        grid=(B // tb,),
        in_specs=[
            pl.BlockSpec((tb, D), lambda i: (i, 0)),
            pl.BlockSpec((D, CKC), lambda i: (0, 0)),
            pl.BlockSpec((D, CKC), lambda i: (0, 0)),
            pl.BlockSpec((1, CK), lambda i: (0, 0)),
            pl.BlockSpec((1, C), lambda i: (0, 0)),
        ],
        out_specs=pl.BlockSpec((2 * C, tb), lambda i: (0, i)),
        out_shape=jax.ShapeDtypeStruct((2 * C, B), jnp.float32),
    )(x, w_a, w_b, const_row, c2_row)


def _sc_stage(tc_out, n_clusters):
    w, B = tc_out.shape
    nc = n_clusters
    info = plsc.get_sparse_core_info()
    nw = info.num_cores * info.num_subcores
    pb = B // nw
    mesh = plsc.VectorSubcoreMesh(core_axis_name="c", subcore_axis_name="s")

    @functools.partial(
        pl.kernel,
        mesh=mesh,
        out_type=jax.ShapeDtypeStruct((B,), jnp.float32),
        scratch_types=[
            pltpu.VMEM((pb * w,), jnp.float32),
            pltpu.VMEM((pb,), jnp.float32),
        ],
        compiler_params=pltpu.CompilerParams(needs_layout_passes=False),
    )
    def sc_kernel(src_hbm, out_hbm, buf_v, res_v):
        wid = lax.axis_index("s") * info.num_cores + lax.axis_index("c")
        base = wid * pb
        for r in range(w):
            pltpu.sync_copy(src_hbm.at[r, pl.ds(base, pb)],
                            buf_v.at[pl.ds(r * pb, pb)])

        def body(i, carry):
            p = i * 16 + jax.lax.iota(jnp.int32, 16)
            besti = jnp.zeros((16,), jnp.int32)
            bestv = plsc.load_gather(buf_v, [p])
            for c in range(1, nc):
                v = plsc.load_gather(buf_v, [c * pb + p])
                m = v < bestv
                bestv = jnp.where(m, v, bestv)
                besti = jnp.where(m, jnp.full((16,), c, jnp.int32), besti)
            ll = plsc.load_gather(buf_v, [(besti + nc) * pb + p])
            res_v[pl.ds(i * 16, 16)] = ll
            return carry

        lax.fori_loop(0, pb // 16, body, 0)
        pltpu.sync_copy(res_v, out_hbm.at[pl.ds(base, pb)])

    return sc_kernel(tc_out)


def kernel(x, centroids, means, log_stds, log_weights):
    B, D = x.shape
    C, K, _ = means.shape
    iv = jnp.exp(-2.0 * log_stds)
    w1 = (means * iv).reshape(C * K, D).T
    w2 = (-0.5 * iv).reshape(C * K, D).T
    w_a = jnp.concatenate([w1, -2.0 * centroids.T],
                          axis=1).astype(jnp.bfloat16)
    w_b = jnp.concatenate(
        [w2, jnp.ones((D, 1), jnp.float32), jnp.zeros((D, C - 1), jnp.float32)],
        axis=1).astype(jnp.bfloat16)
    const_row = ((-0.5 * means * means * iv - log_stds).sum(-1)
                 - 0.5 * D * _LOG2PI
                 + log_weights).reshape(1, C * K).astype(jnp.float32)
    c2_row = (centroids * centroids).sum(-1).reshape(1, C)

    tc_out = _tc_stage(x, w_a, w_b, const_row, c2_row, tb=2048)
    return _sc_stage(tc_out, C)


# speedup vs baseline: 1.1708x; 1.1708x over previous
import functools
import math

import jax
import jax.numpy as jnp
from jax import lax
from jax.experimental import pallas as pl
from jax.experimental.pallas import tpu as pltpu
from jax.experimental.pallas import tpu_sc as plsc

_LOG2PI = math.log(2.0 * math.pi)


def _tc_body(x_ref, wa_ref, wb_ref, const_ref, c2_ref, out_ref):
    x = x_ref[...]
    x2 = x * x
    C = c2_ref.shape[1]
    CK = const_ref.shape[1]
    K = CK // C
    z1 = jnp.dot(x.astype(jnp.bfloat16), wa_ref[...],
                 preferred_element_type=jnp.float32)
    z2 = jnp.dot(x2.astype(jnp.bfloat16), wb_ref[...],
                 preferred_element_type=jnp.float32)
    scores = z2[:, CK:CK + 1] + z1[:, CK:] + c2_ref[...]
    lp = z1[:, :CK] + z2[:, :CK] + const_ref[...]
    lpt = lp.T
    ms, ss = [], []
    for c in range(C):
        seg = lpt[c * K:(c + 1) * K, :]
        m = jnp.max(seg, axis=0, keepdims=True)
        ms.append(m)
        ss.append(jnp.sum(jnp.exp(seg - m), axis=0, keepdims=True))
    lls_t = (jnp.log(jnp.concatenate(ss, axis=0))
             + jnp.concatenate(ms, axis=0))
    out_ref[...] = jnp.concatenate([scores.T, lls_t], axis=0)


def _tc_stage(x, w_a, w_b, const_row, c2_row, tb):
    B, D = x.shape
    CKC = w_a.shape[1]
    C = c2_row.shape[1]
    CK = CKC - C
    return pl.pallas_call(
        _tc_body,
        grid=(B // tb,),
        in_specs=[
            pl.BlockSpec((tb, D), lambda i: (i, 0)),
            pl.BlockSpec((D, CKC), lambda i: (0, 0)),
            pl.BlockSpec((D, CKC), lambda i: (0, 0)),
            pl.BlockSpec((1, CK), lambda i: (0, 0)),
            pl.BlockSpec((1, C), lambda i: (0, 0)),
        ],
        out_specs=pl.BlockSpec((2 * C, tb), lambda i: (0, i)),
        out_shape=jax.ShapeDtypeStruct((2 * C, B), jnp.float32),
    )(x, w_a, w_b, const_row, c2_row)


def _sc_stage(tc_out, n_clusters):
    w, B = tc_out.shape
    nc = n_clusters
    info = plsc.get_sparse_core_info()
    nw = info.num_cores * info.num_subcores
    pb = B // nw
    mesh = plsc.VectorSubcoreMesh(core_axis_name="c", subcore_axis_name="s")

    @functools.partial(
        pl.kernel,
        mesh=mesh,
        out_type=jax.ShapeDtypeStruct((B,), jnp.float32),
        scratch_types=[
            pltpu.VMEM((pb * w,), jnp.float32),
            pltpu.VMEM((pb,), jnp.float32),
            pltpu.SemaphoreType.DMA,
        ],
        compiler_params=pltpu.CompilerParams(needs_layout_passes=False),
    )
    def sc_kernel(src_hbm, out_hbm, buf_v, res_v, sem):
        wid = lax.axis_index("s") * info.num_cores + lax.axis_index("c")
        base = wid * pb
        copies = [pltpu.async_copy(src_hbm.at[r, pl.ds(base, pb)],
                                   buf_v.at[pl.ds(r * pb, pb)], sem)
                  for r in range(w)]
        for cp in copies:
            cp.wait()

        def body(i, carry):
            p = i * 16 + jax.lax.iota(jnp.int32, 16)
            besti = jnp.zeros((16,), jnp.int32)
            bestv = plsc.load_gather(buf_v, [p])
            for c in range(1, nc):
                v = plsc.load_gather(buf_v, [c * pb + p])
                m = v < bestv
                bestv = jnp.where(m, v, bestv)
                besti = jnp.where(m, jnp.full((16,), c, jnp.int32), besti)
            ll = plsc.load_gather(buf_v, [(besti + nc) * pb + p])
            res_v[pl.ds(i * 16, 16)] = ll
            return carry

        lax.fori_loop(0, pb // 16, body, 0)
        pltpu.sync_copy(res_v, out_hbm.at[pl.ds(base, pb)])

    return sc_kernel(tc_out)


def kernel(x, centroids, means, log_stds, log_weights):
    B, D = x.shape
    C, K, _ = means.shape
    iv = jnp.exp(-2.0 * log_stds)
    w1 = (means * iv).reshape(C * K, D).T
    w2 = (-0.5 * iv).reshape(C * K, D).T
    w_a = jnp.concatenate([w1, -2.0 * centroids.T],
                          axis=1).astype(jnp.bfloat16)
    w_b = jnp.concatenate(
        [w2, jnp.ones((D, 1), jnp.float32), jnp.zeros((D, C - 1), jnp.float32)],
        axis=1).astype(jnp.bfloat16)
    const_row = ((-0.5 * means * means * iv - log_stds).sum(-1)
                 - 0.5 * D * _LOG2PI
                 + log_weights).reshape(1, C * K).astype(jnp.float32)
    c2_row = (centroids * centroids).sum(-1).reshape(1, C)

    tc_out = _tc_stage(x, w_a, w_b, const_row, c2_row, tb=2048)
    return _sc_stage(tc_out, C)


# speedup vs baseline: 1.3195x; 1.1270x over previous
import functools
import math

import jax
import jax.numpy as jnp
from jax import lax
from jax.experimental import pallas as pl
from jax.experimental.pallas import tpu as pltpu
from jax.experimental.pallas import tpu_sc as plsc

_LOG2PI = math.log(2.0 * math.pi)
_NT = (((1,), (1,)), ((), ()))


def _tc_body(x_ref, mm_ref, ls_ref, lw_ref, cent_ref,
             out_ref, wa_s, wb_s, cc_s):
    C = cent_ref.shape[0]
    CK = mm_ref.shape[0]
    D = mm_ref.shape[1]
    K = CK // C

    @pl.when(pl.program_id(0) == 0)
    def _prep():
        iv = jnp.exp(-2.0 * ls_ref[...])
        mm = mm_ref[...]
        cent = cent_ref[...]
        wa_s[...] = jnp.concatenate(
            [mm * iv, -2.0 * cent], axis=0).astype(jnp.bfloat16)
        wb_s[...] = jnp.concatenate(
            [-0.5 * iv, jnp.ones((1, D), jnp.float32),
             jnp.zeros((C - 1, D), jnp.float32)],
            axis=0).astype(jnp.bfloat16)
        ones_row = jnp.ones((1, D), jnp.bfloat16)
        cst = lax.dot_general(
            ones_row, (-0.5 * mm * mm * iv - ls_ref[...]).astype(jnp.bfloat16),
            _NT, preferred_element_type=jnp.float32)
        cst = cst + lw_ref[...] - 0.5 * D * _LOG2PI
        c2 = lax.dot_general(ones_row, (cent * cent).astype(jnp.bfloat16),
                             _NT, preferred_element_type=jnp.float32)
        cc_s[...] = jnp.concatenate([cst, c2], axis=1)

    x = x_ref[...]
    x2 = x * x
    z1 = lax.dot_general(x.astype(jnp.bfloat16), wa_s[...], _NT,
                         preferred_element_type=jnp.float32)
    z2 = lax.dot_general(x2.astype(jnp.bfloat16), wb_s[...], _NT,
                         preferred_element_type=jnp.float32)
    cc = cc_s[...]
    scores = z2[:, CK:CK + 1] + z1[:, CK:] + cc[:, CK:]
    lp = z1[:, :CK] + z2[:, :CK] + cc[:, :CK]
    lpt = lp.T
    ms, ss = [], []
    for c in range(C):
        seg = lpt[c * K:(c + 1) * K, :]
        m = jnp.max(seg, axis=0, keepdims=True)
        ms.append(m)
        ss.append(jnp.sum(jnp.exp(seg - m), axis=0, keepdims=True))
    lls_t = (jnp.log(jnp.concatenate(ss, axis=0))
             + jnp.concatenate(ms, axis=0))
    out_ref[...] = jnp.concatenate([scores.T, lls_t], axis=0)


def _tc_stage(x, mm2d, ls2d, lw_row, cent, tb):
    B, D = x.shape
    CK = mm2d.shape[0]
    C = cent.shape[0]
    return pl.pallas_call(
        _tc_body,
        grid=(B // tb,),
        in_specs=[
            pl.BlockSpec((tb, D), lambda i: (i, 0)),
            pl.BlockSpec((CK, D), lambda i: (0, 0)),
            pl.BlockSpec((CK, D), lambda i: (0, 0)),
            pl.BlockSpec((1, CK), lambda i: (0, 0)),
            pl.BlockSpec((C, D), lambda i: (0, 0)),
        ],
        out_specs=pl.BlockSpec((2 * C, tb), lambda i: (0, i)),
        out_shape=jax.ShapeDtypeStruct((2 * C, B), jnp.float32),
        scratch_shapes=[
            pltpu.VMEM((CK + C, D), jnp.bfloat16),
            pltpu.VMEM((CK + C, D), jnp.bfloat16),
            pltpu.VMEM((1, CK + C), jnp.float32),
        ],
    )(x, mm2d, ls2d, lw_row, cent)


def _sc_stage(tc_out, n_clusters):
    w, B = tc_out.shape
    nc = n_clusters
    info = plsc.get_sparse_core_info()
    nw = info.num_cores * info.num_subcores
    pb = B // nw
    mesh = plsc.VectorSubcoreMesh(core_axis_name="c", subcore_axis_name="s")

    @functools.partial(
        pl.kernel,
        mesh=mesh,
        out_type=jax.ShapeDtypeStruct((B,), jnp.float32),
        scratch_types=[
            pltpu.VMEM((pb * w,), jnp.float32),
            pltpu.VMEM((pb,), jnp.float32),
            pltpu.SemaphoreType.DMA,
        ],
        compiler_params=pltpu.CompilerParams(needs_layout_passes=False),
    )
    def sc_kernel(src_hbm, out_hbm, buf_v, res_v, sem):
        wid = lax.axis_index("s") * info.num_cores + lax.axis_index("c")
        base = wid * pb
        copies = [pltpu.async_copy(src_hbm.at[r, pl.ds(base, pb)],
                                   buf_v.at[pl.ds(r * pb, pb)], sem)
                  for r in range(w)]
        for cp in copies:
            cp.wait()

        def body(i, carry):
            p = i * 16 + jax.lax.iota(jnp.int32, 16)
            besti = jnp.zeros((16,), jnp.int32)
            bestv = plsc.load_gather(buf_v, [p])
            for c in range(1, nc):
                v = plsc.load_gather(buf_v, [c * pb + p])
                m = v < bestv
                bestv = jnp.where(m, v, bestv)
                besti = jnp.where(m, jnp.full((16,), c, jnp.int32), besti)
            ll = plsc.load_gather(buf_v, [(besti + nc) * pb + p])
            res_v[pl.ds(i * 16, 16)] = ll
            return carry

        lax.fori_loop(0, pb // 16, body, 0)
        pltpu.sync_copy(res_v, out_hbm.at[pl.ds(base, pb)])

    return sc_kernel(tc_out)


def kernel(x, centroids, means, log_stds, log_weights):
    B, D = x.shape
    C, K, _ = means.shape
    mm2d = means.reshape(C * K, D)
    ls2d = log_stds.reshape(C * K, D)
    lw_row = log_weights.reshape(1, C * K)
    tc_out = _tc_stage(x, mm2d, ls2d, lw_row, centroids, tb=2048)
    return _sc_stage(tc_out, C)
